# Initial kernel scaffold; baseline (speedup 1.0000x reference)
import jax
import jax.numpy as jnp
from jax.experimental import pallas as pl


def kernel(feat, edge_index):
    raise NotImplementedError("write your pallas kernel here")


# speedup vs baseline: 6.1073x; 6.1073x over previous
import functools

import jax
import jax.numpy as jnp
from jax import lax
from jax.experimental import pallas as pl
from jax.experimental.pallas import tpu as pltpu
from jax.experimental.pallas import tpu_sc as plsc

N_NODES = 10000
N_EDGES = 320000
D = 128
H = D // 2
NC = 2
NT = 16
CHUNK = 128
CPT = 158
E_SC = NT * CPT * CHUNK
DUMP_ROW = N_NODES
ACC_ROWS = 10240
ZCHUNKS = ACC_ROWS // NT // CHUNK


def _sc_aggregate(feat_cat, srcs, dsts):
    mesh = plsc.VectorSubcoreMesh(core_axis_name="c", subcore_axis_name="s")

    @functools.partial(
        pl.kernel,
        mesh=mesh,
        out_type=jax.ShapeDtypeStruct((NC, ACC_ROWS, H), jnp.float32),
        compiler_params=pltpu.CompilerParams(use_tc_tiling_on_sc=False),
        scratch_types=[
            pltpu.VMEM((CPT, CHUNK), jnp.int32),
            pltpu.VMEM((CPT, CHUNK), jnp.int32),
            pltpu.VMEM((CHUNK, H), jnp.float32),
            pltpu.VMEM_SHARED((ACC_ROWS, H), jnp.float32),
            pltpu.SemaphoreType.DMA,
        ],
    )
    def body(feat_hbm, src_hbm, dst_hbm, out_hbm, src_v, dst_v, buf_v, acc_sp, sem):
        c = lax.axis_index("c")
        s = lax.axis_index("s")

        pltpu.sync_copy(src_hbm.at[c, s], src_v)
        pltpu.sync_copy(dst_hbm.at[s], dst_v)

        zeros = jnp.zeros((16,), jnp.float32)

        def zero_row(i, carry):
            for j in range(H // 16):
                buf_v[i, pl.ds(j * 16, 16)] = zeros
            return carry

        lax.fori_loop(0, CHUNK, zero_row, 0)
        for k in range(ZCHUNKS):
            pltpu.sync_copy(
                buf_v,
                acc_sp.at[pl.ds(s * (ACC_ROWS // NT) + k * CHUNK, CHUNK)],
            )
        plsc.subcore_barrier()

        def chunk_body(j, carry):
            pltpu.async_copy(feat_hbm.at[src_v.at[j]], buf_v, sem).wait()
            pltpu.sync_copy(buf_v, acc_sp.at[dst_v.at[j]], add=True)
            return carry

        lax.fori_loop(0, CPT, chunk_body, 0)
        plsc.subcore_barrier()

        for k in range(ZCHUNKS):
            r0 = s * (ACC_ROWS // NT) + k * CHUNK
            pltpu.sync_copy(acc_sp.at[pl.ds(r0, CHUNK)], buf_v)
            pltpu.sync_copy(buf_v, out_hbm.at[c, pl.ds(r0, CHUNK)])

    return body(feat_cat, srcs, dsts)


def kernel(feat, edge_index):
    feat = feat.astype(jnp.float32)
    ei = edge_index.astype(jnp.int32)
    src, dst = ei[0], ei[1]

    pad = E_SC - N_EDGES
    pad_src = lax.iota(jnp.int32, pad) % N_NODES
    src_p = jnp.concatenate([src, pad_src])
    dst_p = jnp.concatenate([dst, jnp.full((pad,), DUMP_ROW, jnp.int32)])

    srcs = jnp.stack([src_p, src_p + N_NODES]).reshape(NC, NT, CPT, CHUNK)
    dsts = dst_p.reshape(NT, CPT, CHUNK)
    feat_cat = jnp.concatenate([feat[:, :H], feat[:, H:]], axis=0)

    out2 = _sc_aggregate(feat_cat, srcs, dsts)
    return out2[:, :N_NODES].transpose(1, 0, 2).reshape(N_NODES, D)


# speedup vs baseline: 8.8625x; 1.4511x over previous
import functools

import jax
import jax.numpy as jnp
from jax import lax
from jax.experimental import pallas as pl
from jax.experimental.pallas import tpu as pltpu
from jax.experimental.pallas import tpu_sc as plsc

N_NODES = 10000
N_EDGES = 320000
D = 128
H = D // 2
NC = 2
NT = 16
CHUNK = 128
CPT = 158
E_SC = NT * CPT * CHUNK
DUMP_ROW = N_NODES
ACC_ROWS = 10240
ZCHUNKS = ACC_ROWS // NT // CHUNK


def _sc_aggregate(feat_cat, srcs, dsts):
    mesh = plsc.VectorSubcoreMesh(core_axis_name="c", subcore_axis_name="s")

    @functools.partial(
        pl.kernel,
        mesh=mesh,
        out_type=jax.ShapeDtypeStruct((NC, ACC_ROWS, H), jnp.float32),
        compiler_params=pltpu.CompilerParams(use_tc_tiling_on_sc=False),
        scratch_types=[
            pltpu.VMEM((CPT, CHUNK), jnp.int32),
            pltpu.VMEM((CPT, CHUNK), jnp.int32),
            pltpu.VMEM((CHUNK, H), jnp.float32),
            pltpu.VMEM((CHUNK, H), jnp.float32),
            pltpu.VMEM_SHARED((ACC_ROWS, H), jnp.float32),
            pltpu.SemaphoreType.DMA,
            pltpu.SemaphoreType.DMA,
        ],
    )
    def body(feat_hbm, src_hbm, dst_hbm, out_hbm,
             src_v, dst_v, buf0_v, buf1_v, acc_sp, sem0, sem1):
        buf_v = buf0_v
        c = lax.axis_index("c")
        s = lax.axis_index("s")

        pltpu.sync_copy(src_hbm.at[c, s], src_v)
        pltpu.sync_copy(dst_hbm.at[s], dst_v)

        zeros = jnp.zeros((16,), jnp.float32)

        def zero_row(i, carry):
            for j in range(H // 16):
                buf_v[i, pl.ds(j * 16, 16)] = zeros
            return carry

        lax.fori_loop(0, CHUNK, zero_row, 0)
        for k in range(ZCHUNKS):
            pltpu.sync_copy(
                buf_v,
                acc_sp.at[pl.ds(s * (ACC_ROWS // NT) + k * CHUNK, CHUNK)],
            )
        plsc.subcore_barrier()

        def start_gather(j, buf, sem):
            return pltpu.async_copy(feat_hbm.at[src_v.at[j]], buf, sem)

        def wait_gather(j, buf, sem):
            pltpu.make_async_copy(feat_hbm.at[src_v.at[j]], buf, sem).wait()

        def scatter_add(j, buf):
            pltpu.sync_copy(buf, acc_sp.at[dst_v.at[j]], add=True)

        start_gather(0, buf0_v, sem0)

        def chunk_body(j2, carry):
            j = j2 * 2
            start_gather(j + 1, buf1_v, sem1)
            wait_gather(j, buf0_v, sem0)
            scatter_add(j, buf0_v)
            start_gather(j + 2, buf0_v, sem0)
            wait_gather(j + 1, buf1_v, sem1)
            scatter_add(j + 1, buf1_v)
            return carry

        lax.fori_loop(0, CPT // 2 - 1, chunk_body, 0)
        start_gather(CPT - 1, buf1_v, sem1)
        wait_gather(CPT - 2, buf0_v, sem0)
        scatter_add(CPT - 2, buf0_v)
        wait_gather(CPT - 1, buf1_v, sem1)
        scatter_add(CPT - 1, buf1_v)
        plsc.subcore_barrier()

        for k in range(ZCHUNKS):
            r0 = s * (ACC_ROWS // NT) + k * CHUNK
            pltpu.sync_copy(acc_sp.at[pl.ds(r0, CHUNK)], buf_v)
            pltpu.sync_copy(buf_v, out_hbm.at[c, pl.ds(r0, CHUNK)])

    return body(feat_cat, srcs, dsts)


def kernel(feat, edge_index):
    feat = feat.astype(jnp.float32)
    ei = edge_index.astype(jnp.int32)
    src, dst = ei[0], ei[1]

    pad = E_SC - N_EDGES
    pad_src = lax.iota(jnp.int32, pad) % N_NODES
    src_p = jnp.concatenate([src, pad_src])
    dst_p = jnp.concatenate([dst, jnp.full((pad,), DUMP_ROW, jnp.int32)])

    srcs = jnp.stack([src_p, src_p + N_NODES]).reshape(NC, NT, CPT, CHUNK)
    dsts = dst_p.reshape(NT, CPT, CHUNK)
    feat_cat = jnp.concatenate([feat[:, :H], feat[:, H:]], axis=0)

    out2 = _sc_aggregate(feat_cat, srcs, dsts)
    return out2[:, :N_NODES].transpose(1, 0, 2).reshape(N_NODES, D)
